# Initial kernel scaffold; baseline (speedup 1.0000x reference)
import jax
import jax.numpy as jnp
from jax.experimental import pallas as pl


def kernel(indices, species):
    raise NotImplementedError("write your pallas kernel here")


# speedup vs baseline: 2.7678x; 2.7678x over previous
import functools

import jax
import jax.numpy as jnp
from jax import lax
from jax.experimental import pallas as pl
from jax.experimental.pallas import tpu as pltpu
from jax.experimental.pallas import tpu_sc as plsc

VOCAB = 1026
D = 128
B = 4096 * 50

NC, NS = 2, 16
NW = NC * NS
B_PER_W = B // NW
CHUNK = 256
N_CHUNKS = B_PER_W // CHUNK

_mesh = plsc.VectorSubcoreMesh(core_axis_name="c", subcore_axis_name="s")


@functools.partial(
    pl.kernel,
    mesh=_mesh,
    out_type=jax.ShapeDtypeStruct((B, D), jnp.float32),
    scratch_types=[
        pltpu.VMEM((CHUNK,), jnp.int32),
        pltpu.VMEM((CHUNK, D), jnp.float32),
        pltpu.SemaphoreType.DMA,
    ],
)
def _emb_gather(table_hbm, idx_hbm, out_hbm, idx_v, rows_v, sem):
    wid = lax.axis_index("s") * NC + lax.axis_index("c")
    base = wid * B_PER_W

    def body(i, carry):
        off = base + i * CHUNK
        pltpu.sync_copy(idx_hbm.at[pl.ds(off, CHUNK)], idx_v)
        pltpu.async_copy(table_hbm.at[idx_v], rows_v, sem).wait()
        pltpu.sync_copy(rows_v, out_hbm.at[pl.ds(off, CHUNK)])
        return carry

    lax.fori_loop(0, N_CHUNKS, body, 0)


def kernel(indices, species):
    flat = indices.reshape(B)
    out = _emb_gather(species, flat)
    return out.reshape(indices.shape + (D,))


# speedup vs baseline: 2.7969x; 1.0105x over previous
import functools

import jax
import jax.numpy as jnp
from jax import lax
from jax.experimental import pallas as pl
from jax.experimental.pallas import tpu as pltpu
from jax.experimental.pallas import tpu_sc as plsc

VOCAB = 1026
D = 128
B = 4096 * 50

NC, NS = 2, 16
NW = NC * NS
B_PER_W = B // NW
CHUNK = 400
N_CHUNKS = B_PER_W // CHUNK

_mesh = plsc.VectorSubcoreMesh(core_axis_name="c", subcore_axis_name="s")


@functools.partial(
    pl.kernel,
    mesh=_mesh,
    out_type=jax.ShapeDtypeStruct((B, D), jnp.float32),
    scratch_types=[
        pltpu.VMEM((CHUNK,), jnp.int32),
        pltpu.VMEM((CHUNK,), jnp.int32),
        pltpu.VMEM((CHUNK, D), jnp.float32),
        pltpu.VMEM((CHUNK, D), jnp.float32),
        pltpu.SemaphoreType.DMA,
        pltpu.SemaphoreType.DMA,
        pltpu.SemaphoreType.DMA,
        pltpu.SemaphoreType.DMA,
        pltpu.SemaphoreType.DMA,
        pltpu.SemaphoreType.DMA,
    ],
)
def _emb_gather(table_hbm, idx_hbm, out_hbm,
                idx0, idx1, rows0, rows1,
                si0, si1, sg0, sg1, ss0, ss1):
    wid = lax.axis_index("s") * NC + lax.axis_index("c")
    base = wid * B_PER_W
    idx_v = (idx0, idx1)
    rows_v = (rows0, rows1)
    sem_i = (si0, si1)
    sem_g = (sg0, sg1)
    sem_s = (ss0, ss1)

    def idx_load(c, b):
        cc = jnp.minimum(c, N_CHUNKS - 1)
        pltpu.async_copy(idx_hbm.at[pl.ds(base + cc * CHUNK, CHUNK)],
                         idx_v[b], sem_i[b])

    def gather_start(b):
        pltpu.async_copy(table_hbm.at[idx_v[b]], rows_v[b], sem_g[b])

    def scatter_start(c, b):
        pltpu.async_copy(rows_v[b], out_hbm.at[pl.ds(base + c * CHUNK, CHUNK)],
                         sem_s[b])

    def idx_wait(b):
        pltpu.make_async_copy(idx_hbm.at[pl.ds(0, CHUNK)], idx_v[b],
                              sem_i[b]).wait()

    def gather_wait(b):
        pltpu.make_async_copy(table_hbm.at[idx_v[b]], rows_v[b],
                              sem_g[b]).wait()

    def scatter_wait(b):
        pltpu.make_async_copy(rows_v[b], out_hbm.at[pl.ds(0, CHUNK)],
                              sem_s[b]).wait()

    idx_load(0, 0)
    idx_load(1, 1)
    idx_wait(0)
    gather_start(0)
    idx_wait(1)
    gather_start(1)
    gather_wait(0)
    scatter_start(0, 0)
    idx_load(2, 0)

    def group(g, carry):
        for b in range(2):
            c = 2 * g + 2 + b
            o = 1 - b
            scatter_wait(b)
            idx_wait(b)
            gather_start(b)
            gather_wait(o)
            scatter_start(c - 1, o)
            idx_load(c + 1, o)
        return carry

    lax.fori_loop(0, (N_CHUNKS - 2) // 2, group, 0)

    last = (N_CHUNKS - 1) % 2
    gather_wait(last)
    scatter_start(N_CHUNKS - 1, last)
    scatter_wait(1 - last)
    scatter_wait(last)
    idx_wait(N_CHUNKS % 2)


def kernel(indices, species):
    flat = indices.reshape(B)
    out = _emb_gather(species, flat)
    return out.reshape(indices.shape + (D,))


# speedup vs baseline: 4.4838x; 1.6031x over previous
import functools

import jax
import jax.numpy as jnp
from jax import lax
from jax.experimental import pallas as pl
from jax.experimental.pallas import tpu as pltpu
from jax.experimental.pallas import tpu_sc as plsc

VOCAB = 1026
D = 128
BATCH = 4096
HIST = 50
B = BATCH * HIST

NC, NS = 2, 16
NW = NC * NS
ROWS_PER_W = BATCH // NW
RPC = 8
CHUNK = RPC * HIST
N_CHUNKS = ROWS_PER_W // RPC

_mesh = plsc.VectorSubcoreMesh(core_axis_name="c", subcore_axis_name="s")


@functools.partial(
    pl.kernel,
    mesh=_mesh,
    out_type=jax.ShapeDtypeStruct((BATCH, HIST, D), jnp.float32),
    scratch_types=[
        pltpu.VMEM((CHUNK,), jnp.int32),
        pltpu.VMEM((CHUNK,), jnp.int32),
        pltpu.VMEM((CHUNK, D), jnp.float32),
        pltpu.VMEM((CHUNK, D), jnp.float32),
        pltpu.SemaphoreType.DMA,
        pltpu.SemaphoreType.DMA,
        pltpu.SemaphoreType.DMA,
        pltpu.SemaphoreType.DMA,
        pltpu.SemaphoreType.DMA,
        pltpu.SemaphoreType.DMA,
    ],
)
def _emb_gather(table_hbm, idx_hbm, out_hbm,
                idx0, idx1, rows0, rows1,
                si0, si1, sg0, sg1, ss0, ss1):
    wid = lax.axis_index("s") * NC + lax.axis_index("c")
    base = wid * ROWS_PER_W
    idx_v = (idx0, idx1)
    rows_v = (rows0, rows1)
    sem_i = (si0, si1)
    sem_g = (sg0, sg1)
    sem_s = (ss0, ss1)

    def idx_load(c, b):
        cc = jnp.minimum(c, N_CHUNKS - 1)
        pltpu.async_copy(idx_hbm.at[pl.ds((base + cc * RPC) * HIST, CHUNK)],
                         idx_v[b], sem_i[b])

    def gather_start(b):
        pltpu.async_copy(table_hbm.at[idx_v[b]], rows_v[b], sem_g[b])

    def scatter_start(c, b):
        bo = base + c * RPC
        for j in range(RPC):
            pltpu.async_copy(rows_v[b].at[pl.ds(j * HIST, HIST)],
                             out_hbm.at[bo + j], sem_s[b])

    def idx_wait(b):
        pltpu.make_async_copy(idx_hbm.at[pl.ds(0, CHUNK)], idx_v[b],
                              sem_i[b]).wait()

    def gather_wait(b):
        pltpu.make_async_copy(table_hbm.at[idx_v[b]], rows_v[b],
                              sem_g[b]).wait()

    def scatter_wait(b):
        for j in range(RPC):
            pltpu.make_async_copy(rows_v[b].at[pl.ds(0, HIST)],
                                  out_hbm.at[0], sem_s[b]).wait()

    idx_load(0, 0)
    idx_load(1, 1)
    idx_wait(0)
    gather_start(0)
    idx_wait(1)
    gather_start(1)
    gather_wait(0)
    scatter_start(0, 0)
    idx_load(2, 0)

    def group(g, carry):
        for b in range(2):
            c = 2 * g + 2 + b
            o = 1 - b
            scatter_wait(b)
            idx_wait(b)
            gather_start(b)
            gather_wait(o)
            scatter_start(c - 1, o)
            idx_load(c + 1, o)
        return carry

    lax.fori_loop(0, (N_CHUNKS - 2) // 2, group, 0)

    last = (N_CHUNKS - 1) % 2
    gather_wait(last)
    scatter_start(N_CHUNKS - 1, last)
    scatter_wait(1 - last)
    scatter_wait(last)
    idx_wait(N_CHUNKS % 2)


def kernel(indices, species):
    flat = indices.reshape(B)
    return _emb_gather(species, flat)


# speedup vs baseline: 7.0856x; 1.5803x over previous
import functools

import jax
import jax.numpy as jnp
from jax import lax
from jax.experimental import pallas as pl
from jax.experimental.pallas import tpu as pltpu
from jax.experimental.pallas import tpu_sc as plsc

VOCAB = 1026
D = 128
BATCH = 4096
HIST = 50
B = BATCH * HIST

NC, NS = 2, 16
NW = NC * NS
ROWS_PER_W = BATCH // NW
RPC = 8
CHUNK = RPC * HIST
N_CHUNKS = ROWS_PER_W // RPC

_mesh = plsc.VectorSubcoreMesh(core_axis_name="c", subcore_axis_name="s")


@functools.partial(
    pl.kernel,
    mesh=_mesh,
    out_type=jax.ShapeDtypeStruct((BATCH, HIST, D), jnp.float32),
    scratch_types=[
        pltpu.VMEM((CHUNK,), jnp.int32),
        pltpu.VMEM((CHUNK,), jnp.int32),
        pltpu.VMEM((CHUNK, D), jnp.float32),
        pltpu.VMEM((CHUNK, D), jnp.float32),
        pltpu.VMEM_SHARED((VOCAB, D), jnp.float32),
        pltpu.SemaphoreType.DMA,
        pltpu.SemaphoreType.DMA,
        pltpu.SemaphoreType.DMA,
        pltpu.SemaphoreType.DMA,
        pltpu.SemaphoreType.DMA,
        pltpu.SemaphoreType.DMA,
    ],
)
def _emb_gather(table_hbm, idx_hbm, out_hbm,
                idx0, idx1, rows0, rows1, tab_sh,
                si0, si1, sg0, sg1, ss0, ss1):
    sid = lax.axis_index("s")
    wid = lax.axis_index("s") * NC + lax.axis_index("c")
    base = wid * ROWS_PER_W
    idx_v = (idx0, idx1)
    rows_v = (rows0, rows1)
    sem_i = (si0, si1)
    sem_g = (sg0, sg1)
    sem_s = (ss0, ss1)

    def idx_load(c, b):
        cc = jnp.minimum(c, N_CHUNKS - 1)
        pltpu.async_copy(idx_hbm.at[pl.ds((base + cc * RPC) * HIST, CHUNK)],
                         idx_v[b], sem_i[b])

    def gather_start(b):
        pltpu.async_copy(tab_sh.at[idx_v[b]], rows_v[b], sem_g[b])

    def scatter_start(c, b):
        bo = base + c * RPC
        for j in range(RPC):
            pltpu.async_copy(rows_v[b].at[pl.ds(j * HIST, HIST)],
                             out_hbm.at[bo + j], sem_s[b])

    def idx_wait(b):
        pltpu.make_async_copy(idx_hbm.at[pl.ds(0, CHUNK)], idx_v[b],
                              sem_i[b]).wait()

    def gather_wait(b):
        pltpu.make_async_copy(tab_sh.at[idx_v[b]], rows_v[b],
                              sem_g[b]).wait()

    def scatter_wait(b):
        for j in range(RPC):
            pltpu.make_async_copy(rows_v[b].at[pl.ds(0, HIST)],
                                  out_hbm.at[0], sem_s[b]).wait()

    @pl.when(sid == 0)
    def _():
        pltpu.sync_copy(table_hbm, tab_sh)

    plsc.subcore_barrier()

    idx_load(0, 0)
    idx_load(1, 1)
    idx_wait(0)
    gather_start(0)
    idx_wait(1)
    gather_start(1)
    gather_wait(0)
    scatter_start(0, 0)
    idx_load(2, 0)

    def group(g, carry):
        for b in range(2):
            c = 2 * g + 2 + b
            o = 1 - b
            scatter_wait(b)
            idx_wait(b)
            gather_start(b)
            gather_wait(o)
            scatter_start(c - 1, o)
            idx_load(c + 1, o)
        return carry

    lax.fori_loop(0, (N_CHUNKS - 2) // 2, group, 0)

    last = (N_CHUNKS - 1) % 2
    gather_wait(last)
    scatter_start(N_CHUNKS - 1, last)
    scatter_wait(1 - last)
    scatter_wait(last)
    idx_wait(N_CHUNKS % 2)


def kernel(indices, species):
    flat = indices.reshape(B)
    return _emb_gather(species, flat)
